# Initial kernel scaffold; baseline (speedup 1.0000x reference)
import jax
import jax.numpy as jnp
from jax.experimental import pallas as pl


def kernel(x, Wr, br, We, be):
    raise NotImplementedError("write your pallas kernel here")


# speedup vs baseline: 1.5638x; 1.5638x over previous
import functools

import jax
import jax.numpy as jnp
from jax import lax
from jax.experimental import pallas as pl
from jax.experimental.pallas import tpu as pltpu


def _moe_body(x_ref, wr_ref, br_ref, we_ref, be_ref, out_ref, acc_ref, sel_w_ref):
    e = pl.program_id(1)
    num_e = pl.num_programs(1)
    tm = x_ref.shape[0]

    @pl.when(e == 0)
    def _routing():
        xt = x_ref[...]
        logits = lax.dot_general(xt, wr_ref[...], (((1,), (1,)), ((), ())),
                                 preferred_element_type=jnp.float32)
        logits = logits + br_ref[...]
        max1 = jnp.max(logits, axis=1, keepdims=True)
        expd = jnp.exp(logits - max1)
        probs = expd / jnp.sum(expd, axis=1, keepdims=True)
        iota_e = lax.broadcasted_iota(jnp.int32, logits.shape, 1)
        big = jnp.asarray(num_e, jnp.int32)
        e1 = jnp.min(jnp.where(logits == max1, iota_e, big), axis=1, keepdims=True)
        l2 = jnp.where(iota_e == e1, -jnp.inf, logits)
        max2 = jnp.max(l2, axis=1, keepdims=True)
        e2 = jnp.min(jnp.where(l2 == max2, iota_e, big), axis=1, keepdims=True)
        estar = jnp.maximum(e1, e2)
        w = jnp.sum(jnp.where(iota_e == estar, probs, 0.0), axis=1, keepdims=True)
        sel_w_ref[:, 0:1] = estar.astype(jnp.float32)
        sel_w_ref[:, 1:2] = w

    estar = sel_w_ref[:, 0:1].astype(jnp.int32)
    w = sel_w_ref[:, 1:2]
    sel = estar == e
    val = lax.dot_general(x_ref[...], we_ref[0], (((1,), (1,)), ((), ())),
                          preferred_element_type=jnp.float32)
    val = (val + be_ref[0]) * w

    @pl.when(e == 0)
    def _init():
        acc_ref[...] = jnp.where(sel, val, 0.0)

    @pl.when(e > 0)
    def _update():
        acc_ref[...] = jnp.where(sel, val, acc_ref[...])

    @pl.when(e == num_e - 1)
    def _finish():
        out_ref[...] = acc_ref[...]


def kernel(x, Wr, br, We, be):
    B, S, D = x.shape
    E = Wr.shape[0]
    N = B * S
    x2 = x.reshape(N, D)
    br2 = br.reshape(1, E)
    be3 = be.reshape(E, 1, D)
    TM = 1024
    num_m = N // TM

    out = pl.pallas_call(
        _moe_body,
        grid=(num_m, E),
        in_specs=[
            pl.BlockSpec((TM, D), lambda m, e: (m, 0)),
            pl.BlockSpec((E, D), lambda m, e: (0, 0)),
            pl.BlockSpec((1, E), lambda m, e: (0, 0)),
            pl.BlockSpec((1, D, D), lambda m, e: (e, 0, 0)),
            pl.BlockSpec((1, 1, D), lambda m, e: (e, 0, 0)),
        ],
        out_specs=pl.BlockSpec((TM, D), lambda m, e: (m, 0)),
        out_shape=jax.ShapeDtypeStruct((N, D), jnp.float32),
        scratch_shapes=[
            pltpu.VMEM((TM, D), jnp.float32),
            pltpu.VMEM((TM, 2), jnp.float32),
        ],
        compiler_params=pltpu.CompilerParams(
            dimension_semantics=("parallel", "arbitrary"),
        ),
    )(x2, Wr, br2, We, be3)
    return out.reshape(B, S, D)


# speedup vs baseline: 1.7059x; 1.0909x over previous
import functools

import jax
import jax.numpy as jnp
from jax import lax
from jax.experimental import pallas as pl
from jax.experimental.pallas import tpu as pltpu
from jax.experimental.pallas import tpu_sc as plsc

_TM = 128
_TILE_R = 1024


def _route_body(x_ref, wr_ref, br_ref, estar_ref, w16_ref):
    xt = x_ref[...]
    logits = lax.dot_general(xt, wr_ref[...], (((1,), (1,)), ((), ())),
                             preferred_element_type=jnp.float32)
    logits = logits + br_ref[...]
    max1 = jnp.max(logits, axis=1, keepdims=True)
    ex = jnp.exp(logits - max1)
    probs = ex / jnp.sum(ex, axis=1, keepdims=True)
    iota_e = lax.broadcasted_iota(jnp.int32, logits.shape, 1)
    big = jnp.asarray(logits.shape[1], jnp.int32)
    e1 = jnp.min(jnp.where(logits == max1, iota_e, big), axis=1, keepdims=True)
    l2 = jnp.where(iota_e == e1, -jnp.inf, logits)
    max2 = jnp.max(l2, axis=1, keepdims=True)
    e2 = jnp.min(jnp.where(l2 == max2, iota_e, big), axis=1, keepdims=True)
    es = jnp.maximum(e1, e2)
    w = jnp.sum(jnp.where(iota_e == es, probs, 0.0), axis=1, keepdims=True)
    estar_ref[...] = es
    w16_ref[...] = jnp.broadcast_to(w, (w.shape[0], 128))


def _sort_body(est_ref, pos_ref, te_ref, num_e, tm):
    est = est_ref[...]
    rows, lanes = est.shape
    lane = lax.broadcasted_iota(jnp.int32, (rows, lanes), 1)
    sub = lax.broadcasted_iota(jnp.int32, (rows, 1), 0)
    pos = jnp.zeros((rows, lanes), jnp.int32)
    offs = []
    off = jnp.asarray(0, jnp.int32)
    for e in range(num_e):
        m = (est == e).astype(jnp.int32)
        c = m
        k = 1
        while k < lanes:
            c = c + jnp.where(lane >= k, pltpu.roll(c, k, 1), 0)
            k *= 2
        rowtot = c[:, lanes - 1:lanes]
        r = rowtot
        k = 1
        while k < rows:
            r = r + jnp.where(sub >= k, pltpu.roll(r, k, 0), 0)
            k *= 2
        excl = (c - m) + (r - rowtot)
        pos = pos + jnp.where(m, excl + off, 0)
        offs.append(off)
        cnt = jnp.sum(m)
        off = off + ((cnt + (tm - 1)) // tm) * tm
    bi = (lax.broadcasted_iota(jnp.int32, (8, 128), 0) * 128
          + lax.broadcasted_iota(jnp.int32, (8, 128), 1))
    te = jnp.zeros((8, 128), jnp.int32)
    for e in range(num_e):
        te = te + jnp.where(bi * tm >= offs[e], 1, 0)
    pos_ref[...] = pos
    te_ref[...] = te - 1


def _make_dispatch(n, pad_n, d, nw, chunk):
    rows_w = n // nw
    nch = rows_w // chunk
    mesh = plsc.VectorSubcoreMesh(core_axis_name="c", subcore_axis_name="s")

    @functools.partial(
        pl.kernel, mesh=mesh,
        out_type=(jax.ShapeDtypeStruct((pad_n, d), jnp.float32),
                  jax.ShapeDtypeStruct((pad_n, 128), jnp.float32)),
        scratch_types=[pltpu.VMEM((nch, chunk), jnp.int32),
                       pltpu.VMEM((chunk, d), jnp.float32),
                       pltpu.VMEM((chunk, 128), jnp.float32),
                       pltpu.SemaphoreType.DMA,
                       pltpu.SemaphoreType.DMA],
    )
    def dispatch(x_hbm, w16_hbm, pos_hbm, xs_hbm, ws_hbm,
                 idx_v, xbuf, wbuf, semx, semw):
        cid = lax.axis_index("c")
        sid = lax.axis_index("s")
        wid = sid * 2 + cid
        base = wid * rows_w
        for k in range(nch):
            pltpu.sync_copy(pos_hbm.at[pl.ds(base + k * chunk, chunk)],
                            idx_v.at[k])
        for k in range(nch):
            pltpu.sync_copy(x_hbm.at[pl.ds(base + k * chunk, chunk)], xbuf)
            pltpu.sync_copy(w16_hbm.at[pl.ds(base + k * chunk, chunk)], wbuf)
            cpx = pltpu.async_copy(xbuf, xs_hbm.at[idx_v.at[k]], semx)
            cpw = pltpu.async_copy(wbuf, ws_hbm.at[idx_v.at[k]], semw)
            cpx.wait()
            cpw.wait()

    return dispatch


def _gmm_body(te_ref, xs_ref, ws_ref, we_ref, be_ref, ys_ref):
    del te_ref
    acc = lax.dot_general(xs_ref[...], we_ref[0], (((1,), (1,)), ((), ())),
                          preferred_element_type=jnp.float32)
    ys_ref[...] = (acc + be_ref[0]) * ws_ref[:, 0:1]


def _make_collect(n, pad_n, d, nw, chunk):
    rows_w = n // nw
    mesh = plsc.VectorSubcoreMesh(core_axis_name="c", subcore_axis_name="s")

    @functools.partial(
        pl.kernel, mesh=mesh,
        out_type=jax.ShapeDtypeStruct((n, d), jnp.float32),
        scratch_types=[pltpu.VMEM((rows_w,), jnp.int32),
                       pltpu.VMEM((chunk, d), jnp.float32),
                       pltpu.SemaphoreType.DMA],
    )
    def collect(ys_hbm, pos_hbm, out_hbm, idx_v, buf, sem):
        cid = lax.axis_index("c")
        sid = lax.axis_index("s")
        wid = sid * 2 + cid
        base = wid * rows_w
        pltpu.sync_copy(pos_hbm.at[pl.ds(base, rows_w)], idx_v)
        for k in range(rows_w // chunk):
            cp = pltpu.async_copy(
                ys_hbm.at[idx_v.at[pl.ds(k * chunk, chunk)]], buf, sem)
            cp.wait()
            pltpu.sync_copy(buf, out_hbm.at[pl.ds(base + k * chunk, chunk)])

    return collect


def kernel(x, Wr, br, We, be):
    B, S, D = x.shape
    E = Wr.shape[0]
    N = B * S
    PAD_N = N + E * _TM
    G = PAD_N // _TM
    NW = 32

    x2 = x.reshape(N, D)
    br2 = br.reshape(1, E)
    be3 = be.reshape(E, 1, D)

    estar, w16 = pl.pallas_call(
        _route_body,
        grid=(N // _TILE_R,),
        in_specs=[
            pl.BlockSpec((_TILE_R, D), lambda m: (m, 0)),
            pl.BlockSpec((E, D), lambda m: (0, 0)),
            pl.BlockSpec((1, E), lambda m: (0, 0)),
        ],
        out_specs=[
            pl.BlockSpec((_TILE_R, 1), lambda m: (m, 0)),
            pl.BlockSpec((_TILE_R, 128), lambda m: (m, 0)),
        ],
        out_shape=[
            jax.ShapeDtypeStruct((N, 1), jnp.int32),
            jax.ShapeDtypeStruct((N, 128), jnp.float32),
        ],
        compiler_params=pltpu.CompilerParams(
            dimension_semantics=("parallel",)),
    )(x2, Wr, br2)

    est64 = estar.reshape(N // 128, 128)
    pos64, te8 = pl.pallas_call(
        functools.partial(_sort_body, num_e=E, tm=_TM),
        out_shape=[
            jax.ShapeDtypeStruct((N // 128, 128), jnp.int32),
            jax.ShapeDtypeStruct((8, 128), jnp.int32),
        ],
    )(est64)
    pos_flat = pos64.reshape(N)
    te_flat = te8.reshape(-1)[:G]

    xs, ws = _make_dispatch(N, PAD_N, D, NW, 64)(x2, w16, pos_flat)

    grid_spec = pltpu.PrefetchScalarGridSpec(
        num_scalar_prefetch=1,
        grid=(G,),
        in_specs=[
            pl.BlockSpec((_TM, D), lambda g, te: (g, 0)),
            pl.BlockSpec((_TM, 128), lambda g, te: (g, 0)),
            pl.BlockSpec((1, D, D), lambda g, te: (te[g], 0, 0)),
            pl.BlockSpec((1, 1, D), lambda g, te: (te[g], 0, 0)),
        ],
        out_specs=pl.BlockSpec((_TM, D), lambda g, te: (g, 0)),
    )
    ys = pl.pallas_call(
        _gmm_body,
        grid_spec=grid_spec,
        out_shape=jax.ShapeDtypeStruct((PAD_N, D), jnp.float32),
        compiler_params=pltpu.CompilerParams(
            dimension_semantics=("arbitrary",)),
    )(te_flat, xs, ws, We, be3)

    out2 = _make_collect(N, PAD_N, D, NW, 64)(ys, pos_flat)
    return out2.reshape(B, S, D)
